# Initial kernel scaffold; baseline (speedup 1.0000x reference)
import jax
import jax.numpy as jnp
from jax.experimental import pallas as pl


def kernel(x, upconv_top_index, upconv_down_index):
    raise NotImplementedError("write your pallas kernel here")


# speedup vs baseline: 28.0685x; 28.0685x over previous
import dataclasses
import functools

import jax
import jax.numpy as jnp
from jax import lax
from jax.experimental import pallas as pl
from jax.experimental.pallas import tpu as pltpu
from jax.experimental.pallas import tpu_sc as plsc

RAW = 40962
NEW = RAW * 4 - 6
M = NEW - RAW
BC = 128
NC, NS, L = 2, 16, 16
NW = NC * NS

W_T = 128
T_PW = 1296
T_PAD = NW * T_PW
T_FULL = T_PW // W_T
T_REM = T_PW % W_T

W_D = 64
D_PW = M // NW
D_CHUNKS = D_PW // W_D

_mesh = plsc.VectorSubcoreMesh(core_axis_name="c", subcore_axis_name="s")

_cp = pltpu.CompilerParams()
if "needs_layout_passes" in pltpu.CompilerParams.__dataclass_fields__:
    _cp = dataclasses.replace(_cp, needs_layout_passes=False)


@functools.partial(
    pl.kernel,
    mesh=_mesh,
    compiler_params=_cp,
    out_type=[
        jax.ShapeDtypeStruct((T_PAD, BC), jnp.float32),
        jax.ShapeDtypeStruct((M, BC), jnp.float32),
    ],
    scratch_types=[
        pltpu.VMEM((2 * W_D,), jnp.int32),
        pltpu.VMEM((2 * W_D, BC), jnp.float32),
        pltpu.VMEM((W_D, BC), jnp.float32),
    ],
)
def _sc_gather_mean(xt_hbm, top_hbm, down_hbm, out1_hbm, out2_hbm,
                    idx_v, rows_v, o_v):
    wid = lax.axis_index("s") * NC + lax.axis_index("c")

    def div7(n):
        @pl.loop(0, n // L)
        def _(c):
            sl = pl.ds(c * L, L)
            idx_v[sl] = idx_v[sl] // 7

    def top_chunk(base, n):
        pltpu.sync_copy(top_hbm.at[pl.ds(base, n)], idx_v.at[pl.ds(0, n)])
        div7(n)
        pltpu.sync_copy(xt_hbm.at[idx_v.at[pl.ds(0, n)]],
                        rows_v.at[pl.ds(0, n)])
        pltpu.sync_copy(rows_v.at[pl.ds(0, n)], out1_hbm.at[pl.ds(base, n)])

    tbase = wid * T_PW

    @pl.loop(0, T_FULL)
    def _(ci):
        top_chunk(tbase + ci * W_T, W_T)

    top_chunk(tbase + T_FULL * W_T, T_REM)

    dbase = wid * D_PW

    @pl.loop(0, D_CHUNKS)
    def _(ci):
        pbase = dbase + ci * W_D
        pltpu.sync_copy(down_hbm.at[pl.ds(2 * pbase, 2 * W_D)], idx_v)
        div7(2 * W_D)
        pltpu.sync_copy(xt_hbm.at[idx_v], rows_v)

        @pl.loop(0, W_D)
        def _(i):
            @pl.loop(0, BC // L)
            def _(c):
                sl = pl.ds(c * L, L)
                o_v[i, sl] = (rows_v[2 * i, sl] + rows_v[2 * i + 1, sl]) * 0.5

        pltpu.sync_copy(o_v, out2_hbm.at[pl.ds(pbase, W_D)])


def kernel(x, upconv_top_index, upconv_down_index):
    B, C, R = x.shape
    xt = x.reshape(B * C, R).T
    top_pad = jnp.pad(upconv_top_index, (0, T_PAD - R))
    out1, out2 = _sc_gather_mean(xt, top_pad, upconv_down_index)
    o1 = out1[:R].T.reshape(B, C, R)
    o2 = out2.T.reshape(B, C, M)
    return jnp.concatenate([o1, o2], axis=2)


# speedup vs baseline: 35.5701x; 1.2673x over previous
import dataclasses
import functools

import jax
import jax.numpy as jnp
from jax import lax
from jax.experimental import pallas as pl
from jax.experimental.pallas import tpu as pltpu
from jax.experimental.pallas import tpu_sc as plsc

RAW = 40962
NEW = RAW * 4 - 6
M = NEW - RAW
BC = 128
NC, NS, L = 2, 16, 16
NW = NC * NS

W_T = 128
T_PW = 1296
T_PAD = NW * T_PW
T_FULL = T_PW // W_T
T_REM = T_PW % W_T

W_D = 64
D_PW = M // NW
D_CHUNKS = D_PW // W_D

_mesh = plsc.VectorSubcoreMesh(core_axis_name="c", subcore_axis_name="s")

_cp = pltpu.CompilerParams()
if "needs_layout_passes" in pltpu.CompilerParams.__dataclass_fields__:
    _cp = dataclasses.replace(_cp, needs_layout_passes=False)


@functools.partial(
    pl.kernel,
    mesh=_mesh,
    compiler_params=_cp,
    out_type=[
        jax.ShapeDtypeStruct((T_PAD, BC), jnp.float32),
        jax.ShapeDtypeStruct((M, BC), jnp.float32),
    ],
    scratch_types=[
        pltpu.VMEM((T_PW,), jnp.int32),
        pltpu.VMEM((2 * D_PW,), jnp.int32),
        pltpu.VMEM((2 * W_D, BC), jnp.float32),
        pltpu.VMEM((2 * W_D, BC), jnp.float32),
        pltpu.VMEM((W_D, BC), jnp.float32),
        pltpu.VMEM((W_D, BC), jnp.float32),
        pltpu.SemaphoreType.DMA,
        pltpu.SemaphoreType.DMA,
        pltpu.SemaphoreType.DMA,
        pltpu.SemaphoreType.DMA,
    ],
)
def _sc_gather_mean(xt_hbm, top_hbm, down_hbm, out1_hbm, out2_hbm,
                    idx_t, idx_d, rows0, rows1, o0, o1, g0, g1, s0, s1):
    wid = lax.axis_index("s") * NC + lax.axis_index("c")
    rows = (rows0, rows1)
    o = (o0, o1)
    gs = (g0, g1)
    ss = (s0, s1)
    tbase = wid * T_PW
    dbase = wid * D_PW

    pltpu.sync_copy(top_hbm.at[pl.ds(tbase, T_PW)], idx_t)
    pltpu.sync_copy(down_hbm.at[pl.ds(2 * dbase, 2 * D_PW)], idx_d)

    @pl.loop(0, T_PW // L)
    def _(k):
        sl = pl.ds(k * L, L)
        idx_t[sl] = idx_t[sl] // 7

    @pl.loop(0, (2 * D_PW) // L)
    def _(k):
        sl = pl.ds(k * L, L)
        idx_d[sl] = idx_d[sl] // 7

    def tg(c, b):
        pltpu.async_copy(xt_hbm.at[idx_t.at[pl.ds(c * W_T, W_T)]],
                         rows[b], gs[b])

    def twait(b):
        pltpu.make_async_copy(xt_hbm.at[idx_t.at[pl.ds(0, W_T)]],
                              rows[b], gs[b]).wait()

    def tstore(c, b):
        pltpu.sync_copy(rows[b], out1_hbm.at[pl.ds(tbase + c * W_T, W_T)])

    tg(0, 0)
    tg(1, 1)
    twait(0); tstore(0, 0); tg(2, 0)
    twait(1); tstore(1, 1); tg(3, 1)

    @pl.loop(2, T_FULL - 2, step=2)
    def _(ci):
        for b in range(2):
            c = ci + b
            twait(b)
            tstore(c, b)
            tg(c + 2, b)

    twait(0); tstore(T_FULL - 2, 0)
    twait(1); tstore(T_FULL - 1, 1)
    pltpu.sync_copy(xt_hbm.at[idx_t.at[pl.ds(T_FULL * W_T, T_REM)]],
                    rows0.at[pl.ds(0, T_REM)])
    pltpu.sync_copy(rows0.at[pl.ds(0, T_REM)],
                    out1_hbm.at[pl.ds(tbase + T_FULL * W_T, T_REM)])

    def dg(c, b):
        pltpu.async_copy(xt_hbm.at[idx_d.at[pl.ds(c * 2 * W_D, 2 * W_D)]],
                         rows[b], gs[b])

    def dwait(b):
        pltpu.make_async_copy(xt_hbm.at[idx_d.at[pl.ds(0, 2 * W_D)]],
                              rows[b], gs[b]).wait()

    def dcompute(b):
        @pl.loop(0, W_D)
        def _(i):
            for k in range(BC // L):
                sl = pl.ds(k * L, L)
                o[b][i, sl] = (rows[b][2 * i, sl] + rows[b][2 * i + 1, sl]) * 0.5

    def dstore(c, b):
        pltpu.async_copy(o[b], out2_hbm.at[pl.ds(dbase + c * W_D, W_D)], ss[b])

    def dswait(b):
        pltpu.make_async_copy(o[b], out2_hbm.at[pl.ds(0, W_D)], ss[b]).wait()

    dg(0, 0)
    dg(1, 1)
    dwait(0); dcompute(0); dg(2, 0); dstore(0, 0)
    dwait(1); dcompute(1); dg(3, 1); dstore(1, 1)

    @pl.loop(2, D_CHUNKS - 2, step=2)
    def _(ci):
        for b in range(2):
            c = ci + b
            dwait(b)
            dswait(b)
            dcompute(b)
            dg(c + 2, b)
            dstore(c, b)

    for b, c in ((0, D_CHUNKS - 2), (1, D_CHUNKS - 1)):
        dwait(b)
        dswait(b)
        dcompute(b)
        dstore(c, b)
    dswait(0)
    dswait(1)


def kernel(x, upconv_top_index, upconv_down_index):
    B, C, R = x.shape
    xt = x.reshape(B * C, R).T
    top_pad = jnp.pad(upconv_top_index, (0, T_PAD - R))
    out1, out2 = _sc_gather_mean(xt, top_pad, upconv_down_index)
    o1 = out1[:R].T.reshape(B, C, R)
    o2 = out2.T.reshape(B, C, M)
    return jnp.concatenate([o1, o2], axis=2)
